# baseline (speedup 1.0000x reference)
import functools

import jax
import jax.numpy as jnp
from jax import lax
from jax.experimental import pallas as pl
from jax.experimental.pallas import tpu as pltpu
from jax.experimental.pallas import tpu_sc as plsc

SOM0, SOM1 = 64, 128
KN = SOM0 * SOM1
LATENT = 256
IN_DIM = 1024
BATCH = 4096

TB1 = 512
TB2 = 512

_NC, _NS, _L = 2, 16, 16
_NW = _NC * _NS
_BPW = BATCH // _NW


def _stage1_body(x_ref, we_ref, be_ref, embt_ref, wde_ref, bde_ref,
                 zd_ref, k_ref, ze_ref, xe_ref, e2_ref):
    @pl.when(pl.program_id(0) == 0)
    def _():
        et = embt_ref[...]
        e2_ref[...] = jnp.sum(et * et, axis=0, keepdims=True)

    z_e = jnp.dot(x_ref[...], we_ref[...],
                  preferred_element_type=jnp.float32) + be_ref[...]
    ze_ref[...] = z_e
    xe_ref[...] = jnp.dot(z_e, wde_ref[...],
                          preferred_element_type=jnp.float32) + bde_ref[...]
    c = jnp.dot(z_e, embt_ref[...], preferred_element_type=jnp.float32)
    z2 = jnp.sum(z_e * z_e, axis=1, keepdims=True)
    dist = z2 - 2.0 * c + e2_ref[...]
    zd_ref[...] = dist
    k_ref[...] = jnp.argmin(dist, axis=1, keepdims=True).astype(jnp.int32)


_stage1 = pl.pallas_call(
    _stage1_body,
    grid=(BATCH // TB1,),
    in_specs=[
        pl.BlockSpec((TB1, IN_DIM), lambda i: (i, 0)),
        pl.BlockSpec((IN_DIM, LATENT), lambda i: (0, 0)),
        pl.BlockSpec((1, LATENT), lambda i: (0, 0)),
        pl.BlockSpec((LATENT, KN), lambda i: (0, 0)),
        pl.BlockSpec((LATENT, IN_DIM), lambda i: (0, 0)),
        pl.BlockSpec((1, IN_DIM), lambda i: (0, 0)),
    ],
    out_specs=[
        pl.BlockSpec((TB1, KN), lambda i: (i, 0)),
        pl.BlockSpec((TB1, 1), lambda i: (i, 0)),
        pl.BlockSpec((TB1, LATENT), lambda i: (i, 0)),
        pl.BlockSpec((TB1, IN_DIM), lambda i: (i, 0)),
    ],
    out_shape=[
        jax.ShapeDtypeStruct((BATCH, KN), jnp.float32),
        jax.ShapeDtypeStruct((BATCH, 1), jnp.int32),
        jax.ShapeDtypeStruct((BATCH, LATENT), jnp.float32),
        jax.ShapeDtypeStruct((BATCH, IN_DIM), jnp.float32),
    ],
    scratch_shapes=[pltpu.VMEM((1, KN), jnp.float32)],
)


_CH = 32
_NCH = _BPW // _CH
_NB = 8


_TASKS = [(s, c) for s in range(5) for c in range(_NCH)]


def _gather_sc_body(emb_hbm, k_hbm, zq_hbm, nb_hbm, idx_all, *rest):
    bufs = list(rest[:_NB])
    gsem = list(rest[_NB:2 * _NB])
    ssem = list(rest[2 * _NB:3 * _NB])
    wid = lax.axis_index("s") * _NC + lax.axis_index("c")
    base = wid * _BPW

    pltpu.sync_copy(k_hbm.at[pl.ds(base, _BPW)], idx_all.at[pl.ds(0, _BPW)])
    for j in range(_BPW // _L):
        sl = pl.ds(j * _L, _L)
        ci = idx_all[sl]
        k1 = ci >> 7
        k2 = ci & (SOM1 - 1)
        pad = KN + j * _L + lax.iota(jnp.int32, _L)
        idx_all[pl.ds(_BPW + j * _L, _L)] = jnp.where(
            k1 < SOM0 - 1, ci + SOM1, pad)
        idx_all[pl.ds(2 * _BPW + j * _L, _L)] = jnp.where(k1 > 0, ci - SOM1, pad)
        idx_all[pl.ds(3 * _BPW + j * _L, _L)] = pad
        idx_all[pl.ds(4 * _BPW + j * _L, _L)] = jnp.where(k2 > 0, ci - 1, pad)

    n_tasks = len(_TASKS)

    def fire_gather(t):
        s, c = _TASKS[t]
        b = t % _NB
        idxs = idx_all.at[pl.ds(s * _BPW + c * _CH, _CH)]
        return pltpu.async_copy(emb_hbm.at[idxs], bufs[b], gsem[b])

    def fire_scatter(t):
        s, c = _TASKS[t]
        b = t % _NB
        row0 = base + c * _CH
        out = [pltpu.async_copy(bufs[b], nb_hbm.at[s, pl.ds(row0, _CH), :],
                                ssem[b])]
        if s == 0:
            out.append(pltpu.async_copy(bufs[b], zq_hbm.at[pl.ds(row0, _CH)],
                                        ssem[b]))
        return out

    gdesc = [None] * _NB
    pend = [[] for _ in range(_NB)]
    for t in range(min(_NB, n_tasks)):
        gdesc[t] = fire_gather(t)
    for t in range(n_tasks):
        b = t % _NB
        gdesc[b].wait()
        pend[b] = fire_scatter(t)
        nt = t + _NB
        if nt < n_tasks:
            for d in pend[b]:
                d.wait()
            pend[b] = []
            gdesc[b] = fire_gather(nt)
    for b in range(_NB):
        for d in pend[b]:
            d.wait()


@functools.cache
def _gather_sc():
    return pl.kernel(
        _gather_sc_body,
        mesh=plsc.VectorSubcoreMesh(core_axis_name="c", subcore_axis_name="s"),
        out_type=(jax.ShapeDtypeStruct((BATCH, LATENT), jnp.float32),
                  jax.ShapeDtypeStruct((5, BATCH, LATENT), jnp.float32)),
        scratch_types=(
            [pltpu.VMEM((5 * _BPW,), jnp.int32)]
            + [pltpu.VMEM((_CH, LATENT), jnp.float32) for _ in range(_NB)]
            + [pltpu.SemaphoreType.DMA for _ in range(2 * _NB)]
        ),
    )


def _stage3_body(zq_ref, nb5_ref, wdq_ref, bdq_ref, xq_ref, nb_ref):
    xq_ref[...] = jnp.dot(zq_ref[...], wdq_ref[...],
                          preferred_element_type=jnp.float32) + bdq_ref[...]
    for s in range(5):
        nb_ref[:, s, :] = nb5_ref[s, :, :]


_stage3 = pl.pallas_call(
    _stage3_body,
    grid=(BATCH // TB2,),
    in_specs=[
        pl.BlockSpec((TB2, LATENT), lambda i: (i, 0)),
        pl.BlockSpec((5, TB2, LATENT), lambda i: (0, i, 0)),
        pl.BlockSpec((LATENT, IN_DIM), lambda i: (0, 0)),
        pl.BlockSpec((1, IN_DIM), lambda i: (0, 0)),
    ],
    out_specs=[
        pl.BlockSpec((TB2, IN_DIM), lambda i: (i, 0)),
        pl.BlockSpec((TB2, 5, LATENT), lambda i: (i, 0, 0)),
    ],
    out_shape=[
        jax.ShapeDtypeStruct((BATCH, IN_DIM), jnp.float32),
        jax.ShapeDtypeStruct((BATCH, 5, LATENT), jnp.float32),
    ],
)


def kernel(x, W_enc, b_enc, embeddings, W_dq, b_dq, W_de, b_de):
    emb_flat = embeddings.reshape(KN, LATENT)
    embt = emb_flat.T
    zd, k2d, z_e, x_e = _stage1(x, W_enc, b_enc.reshape(1, LATENT), embt,
                                W_de, b_de.reshape(1, IN_DIM))
    k = k2d.reshape(BATCH)
    emb_pad = jnp.concatenate(
        [emb_flat, jnp.zeros((_BPW, LATENT), jnp.float32)], axis=0)
    z_q, nb5 = _gather_sc()(emb_pad, k)
    x_q, z_q_neighbors = _stage3(z_q, nb5, W_dq, b_dq.reshape(1, IN_DIM))
    return (x_e, x_q, z_e, z_q, z_q_neighbors, k, zd)


# speedup vs baseline: 1.2815x; 1.2815x over previous
import functools

import jax
import jax.numpy as jnp
from jax import lax
from jax.experimental import pallas as pl
from jax.experimental.pallas import tpu as pltpu
from jax.experimental.pallas import tpu_sc as plsc

SOM0, SOM1 = 64, 128
KN = SOM0 * SOM1
LATENT = 256
IN_DIM = 1024
BATCH = 4096

TB1 = 512
TB2 = 512

_NC, _NS, _L = 2, 16, 16
_NW = _NC * _NS
_BPW = BATCH // _NW


def _stage1_body(x_ref, we_ref, be_ref, embt_ref, wde_ref, bde_ref,
                 zd_ref, k_ref, ze_ref, xe_ref, e2_ref):
    @pl.when(pl.program_id(0) == 0)
    def _():
        et = embt_ref[...]
        e2_ref[...] = jnp.sum(et * et, axis=0, keepdims=True)

    z_e = jnp.dot(x_ref[...], we_ref[...],
                  preferred_element_type=jnp.float32) + be_ref[...]
    ze_ref[...] = z_e
    xe_ref[...] = jnp.dot(z_e, wde_ref[...],
                          preferred_element_type=jnp.float32) + bde_ref[...]
    c = jnp.dot(z_e, embt_ref[...], preferred_element_type=jnp.float32)
    z2 = jnp.sum(z_e * z_e, axis=1, keepdims=True)
    dist = z2 - 2.0 * c + e2_ref[...]
    zd_ref[...] = dist
    k_ref[...] = jnp.argmin(dist, axis=1, keepdims=True).astype(jnp.int32)


_stage1 = pl.pallas_call(
    _stage1_body,
    grid=(BATCH // TB1,),
    in_specs=[
        pl.BlockSpec((TB1, IN_DIM), lambda i: (i, 0)),
        pl.BlockSpec((IN_DIM, LATENT), lambda i: (0, 0)),
        pl.BlockSpec((1, LATENT), lambda i: (0, 0)),
        pl.BlockSpec((LATENT, KN), lambda i: (0, 0)),
        pl.BlockSpec((LATENT, IN_DIM), lambda i: (0, 0)),
        pl.BlockSpec((1, IN_DIM), lambda i: (0, 0)),
    ],
    out_specs=[
        pl.BlockSpec((TB1, KN), lambda i: (i, 0)),
        pl.BlockSpec((TB1, 1), lambda i: (i, 0)),
        pl.BlockSpec((TB1, LATENT), lambda i: (i, 0)),
        pl.BlockSpec((TB1, IN_DIM), lambda i: (i, 0)),
    ],
    out_shape=[
        jax.ShapeDtypeStruct((BATCH, KN), jnp.float32),
        jax.ShapeDtypeStruct((BATCH, 1), jnp.int32),
        jax.ShapeDtypeStruct((BATCH, LATENT), jnp.float32),
        jax.ShapeDtypeStruct((BATCH, IN_DIM), jnp.float32),
    ],
    scratch_shapes=[pltpu.VMEM((1, KN), jnp.float32)],
)


_CH = 32
_NCH = _BPW // _CH
_NB = 8


def _run_pipeline(n_tasks, nb, fire_gather, fire_scatter):
    gdesc = [None] * nb
    pend = [[] for _ in range(nb)]
    for t in range(min(nb, n_tasks)):
        gdesc[t] = fire_gather(t)
    for t in range(n_tasks):
        b = t % nb
        for d in gdesc[b]:
            d.wait()
        pend[b] = fire_scatter(t)
        nt = t + nb
        if nt < n_tasks:
            for d in pend[b]:
                d.wait()
            pend[b] = []
            gdesc[b] = fire_gather(nt)
    for ds in pend:
        for d in ds:
            d.wait()


def _gather_sc_body(emb_hbm, k_hbm, zq_hbm, nb_hbm, idx_all, *rest):
    bufs = list(rest[:_NB])
    gsem = list(rest[_NB:2 * _NB])
    ssem = list(rest[2 * _NB:3 * _NB])
    wid = lax.axis_index("s") * _NC + lax.axis_index("c")
    base = wid * _BPW

    pltpu.sync_copy(k_hbm.at[pl.ds(base, _BPW)], idx_all.at[pl.ds(0, _BPW)])
    for j in range(_BPW // _L):
        sl = pl.ds(j * _L, _L)
        ci = idx_all[sl]
        k1 = ci >> 7
        k2 = ci & (SOM1 - 1)
        pad = KN + j * _L + lax.iota(jnp.int32, _L)
        idx_all[pl.ds(_BPW + j * _L, _L)] = jnp.where(
            k1 < SOM0 - 1, ci + SOM1, pad)
        idx_all[pl.ds(2 * _BPW + j * _L, _L)] = jnp.where(k1 > 0, ci - SOM1, pad)
        idx_all[pl.ds(3 * _BPW + j * _L, _L)] = pad
        idx_all[pl.ds(4 * _BPW + j * _L, _L)] = jnp.where(k2 > 0, ci - 1, pad)

    tasks = [(s, c) for s in range(5) for c in range(_NCH)]

    def fire_gather(t):
        s, c = tasks[t]
        b = t % _NB
        idxs = idx_all.at[pl.ds(s * _BPW + c * _CH, _CH)]
        return [pltpu.async_copy(emb_hbm.at[idxs], bufs[b], gsem[b])]

    def fire_scatter(t):
        s, c = tasks[t]
        b = t % _NB
        row0 = base + c * _CH
        out = [pltpu.async_copy(bufs[b], nb_hbm.at[s, pl.ds(row0, _CH), :],
                                ssem[b])]
        if s == 0:
            out.append(pltpu.async_copy(bufs[b], zq_hbm.at[pl.ds(row0, _CH)],
                                        ssem[b]))
        return out

    _run_pipeline(len(tasks), _NB, fire_gather, fire_scatter)


@functools.cache
def _gather_sc():
    return pl.kernel(
        _gather_sc_body,
        mesh=plsc.VectorSubcoreMesh(core_axis_name="c", subcore_axis_name="s"),
        out_type=(jax.ShapeDtypeStruct((BATCH, LATENT), jnp.float32),
                  jax.ShapeDtypeStruct((5, BATCH, LATENT), jnp.float32)),
        scratch_types=(
            [pltpu.VMEM((5 * _BPW,), jnp.int32)]
            + [pltpu.VMEM((_CH, LATENT), jnp.float32) for _ in range(_NB)]
            + [pltpu.SemaphoreType.DMA for _ in range(2 * _NB)]
        ),
    )


def _stage3_body(zq_ref, wdq_ref, bdq_ref, xq_ref):
    xq_ref[...] = jnp.dot(zq_ref[...], wdq_ref[...],
                          preferred_element_type=jnp.float32) + bdq_ref[...]


_stage3 = pl.pallas_call(
    _stage3_body,
    grid=(BATCH // TB2,),
    in_specs=[
        pl.BlockSpec((TB2, LATENT), lambda i: (i, 0)),
        pl.BlockSpec((LATENT, IN_DIM), lambda i: (0, 0)),
        pl.BlockSpec((1, IN_DIM), lambda i: (0, 0)),
    ],
    out_specs=pl.BlockSpec((TB2, IN_DIM), lambda i: (i, 0)),
    out_shape=jax.ShapeDtypeStruct((BATCH, IN_DIM), jnp.float32),
)


def kernel(x, W_enc, b_enc, embeddings, W_dq, b_dq, W_de, b_de):
    emb_flat = embeddings.reshape(KN, LATENT)
    embt = emb_flat.T
    zd, k2d, z_e, x_e = _stage1(x, W_enc, b_enc.reshape(1, LATENT), embt,
                                W_de, b_de.reshape(1, IN_DIM))
    k = k2d.reshape(BATCH)
    emb_pad = jnp.concatenate(
        [emb_flat, jnp.zeros((_BPW, LATENT), jnp.float32)], axis=0)
    z_q, nb5 = _gather_sc()(emb_pad, k)
    z_q_neighbors = jnp.transpose(nb5, (1, 0, 2))
    x_q = _stage3(z_q, W_dq, b_dq.reshape(1, IN_DIM))
    return (x_e, x_q, z_e, z_q, z_q_neighbors, k, zd)
